# baseline (speedup 1.0000x reference)
import jax
import jax.numpy as jnp
from jax.experimental import pallas as pl


def _fused_kernel(xb_ref, w_ref, wg_ref, thr_ref, bias_ref, o_ref, coef_ref):
    j = pl.program_id(0)
    C = w_ref.shape[1]

    @pl.when(j == 0)
    def _gating():
        logits = jnp.dot(xb_ref[...], wg_ref[...],
                         preferred_element_type=jnp.float32)
        g = (logits > thr_ref[...]).astype(jnp.float32)
        act = float(C) * jnp.sum(g, axis=1, keepdims=True)
        denom = jnp.where(act > 0.0, act, 1.0)
        coef_ref[...] = g * (float(C) / denom)

    acc = jax.lax.dot_general(
        xb_ref[...], w_ref[0], (((1,), (1,)), ((), ())),
        preferred_element_type=jnp.float32)
    E = coef_ref.shape[1]
    onehot = (jax.lax.broadcasted_iota(jnp.int32, (1, E), 1) == j)
    c = jnp.sum(jnp.where(onehot, coef_ref[...], 0.0),
                axis=1, keepdims=True)
    o_ref[...] = (acc + bias_ref[0]) * c


def kernel(x, Wg_w, Wg_b, comp_w, comp_b):
    B, D = x.shape
    E, C, _ = comp_w.shape

    U = jax.random.uniform(jax.random.key(42), (B, E), jnp.float32)
    thr = jnp.log(U) - jnp.log1p(-U) - Wg_b[None, :]

    xb = x.astype(jnp.bfloat16)
    wb = comp_w.astype(jnp.bfloat16)
    wg = Wg_w.T.astype(jnp.bfloat16)
    bias3 = comp_b[:, None, :]

    out, coef = pl.pallas_call(
        _fused_kernel,
        grid=(E,),
        in_specs=[
            pl.BlockSpec((B, D), lambda j: (0, 0)),
            pl.BlockSpec((1, C, D), lambda j: (j, 0, 0)),
            pl.BlockSpec((D, E), lambda j: (0, 0)),
            pl.BlockSpec((B, E), lambda j: (0, 0)),
            pl.BlockSpec((1, 1, C), lambda j: (j, 0, 0)),
        ],
        out_specs=[
            pl.BlockSpec((B, C), lambda j: (0, j)),
            pl.BlockSpec((B, E), lambda j: (0, 0)),
        ],
        out_shape=[
            jax.ShapeDtypeStruct((B, E * C), jnp.float32),
            jax.ShapeDtypeStruct((B, E), jnp.float32),
        ],
    )(xb, wb, wg, thr, bias3)
    G = (coef > 0.0).astype(jnp.float32)
    return out, G


# speedup vs baseline: 1.2267x; 1.2267x over previous
import jax
import jax.numpy as jnp
from jax.experimental import pallas as pl


def _fused_kernel(xb_ref, w_ref, wg_ref, thr_ref, bias_ref, o_ref, coef_ref):
    j = pl.program_id(0)
    C = w_ref.shape[1]

    @pl.when(j == 0)
    def _gating():
        logits = jax.lax.dot_general(
            xb_ref[...], wg_ref[...], (((1,), (1,)), ((), ())),
            preferred_element_type=jnp.float32)
        thr = jnp.transpose(thr_ref[...])
        g = (logits > thr).astype(jnp.float32)
        act = float(C) * jnp.sum(g, axis=1, keepdims=True)
        denom = jnp.where(act > 0.0, act, 1.0)
        coef_ref[...] = g * (float(C) / denom)

    w = w_ref[0].astype(jnp.bfloat16)
    acc = jax.lax.dot_general(
        xb_ref[...], w, (((1,), (1,)), ((), ())),
        preferred_element_type=jnp.float32)
    E = coef_ref.shape[1]
    onehot = (jax.lax.broadcasted_iota(jnp.int32, (1, E), 1) == j)
    c = jnp.sum(jnp.where(onehot, coef_ref[...], 0.0),
                axis=1, keepdims=True)
    o_ref[...] = (acc + bias_ref[0]) * c


def kernel(x, Wg_w, Wg_b, comp_w, comp_b):
    B, D = x.shape
    E, C, _ = comp_w.shape

    U = jax.random.uniform(jax.random.key(42), (B, E), jnp.float32)
    thrT = (jnp.log(U) - jnp.log1p(-U) - Wg_b[None, :]).T

    xb = x.astype(jnp.bfloat16)
    wg = Wg_w.astype(jnp.bfloat16)
    bias3 = comp_b[:, None, :]

    out, coef = pl.pallas_call(
        _fused_kernel,
        grid=(E,),
        in_specs=[
            pl.BlockSpec((B, D), lambda j: (0, 0)),
            pl.BlockSpec((1, C, D), lambda j: (j, 0, 0)),
            pl.BlockSpec((E, D), lambda j: (0, 0)),
            pl.BlockSpec((E, B), lambda j: (0, 0)),
            pl.BlockSpec((1, 1, C), lambda j: (j, 0, 0)),
        ],
        out_specs=[
            pl.BlockSpec((B, C), lambda j: (0, j)),
            pl.BlockSpec((B, E), lambda j: (0, 0)),
        ],
        out_shape=[
            jax.ShapeDtypeStruct((B, E * C), jnp.float32),
            jax.ShapeDtypeStruct((B, E), jnp.float32),
        ],
    )(xb, comp_w, wg, thrT, bias3)
    G = (coef > 0.0).astype(jnp.float32)
    return out, G
